# baseline (speedup 1.0000x reference)
import functools

import jax
import jax.numpy as jnp
from jax.experimental import pallas as pl
from jax.experimental.pallas import tpu as pltpu

_EPS = 1e-5
_CH = 32


def _fused(x_hbm, mb_ref, wt_ref, atw_ref, gam_ref, bet_ref,
           out_hbm, hsc, zsc, iobuf, st, aff, insem, outsem,
           *, B, T, V, C):
    pid = pl.program_id(0)
    VC = V * C
    NCH = T // _CH
    n = float(B * T * V)

    def x_copy(g, slot):
        return pltpu.make_async_copy(x_hbm.at[g], iobuf.at[slot], insem.at[slot])

    def o_copy(g, slot):
        return pltpu.make_async_copy(iobuf.at[slot], out_hbm.at[g], outsem.at[slot])

    def fold(row):
        acc = row[:, 0:C]
        for v in range(1, V):
            acc = acc + row[:, v * C:(v + 1) * C]
        return acc

    def finalize(srow, qrow, dst):
        mean = fold(st[srow:srow + 1, :]) / n
        var = fold(st[qrow:qrow + 1, :]) / n - mean * mean
        inv = gam_ref[...] * jax.lax.rsqrt(var + _EPS)
        aff[dst:dst + 1, :] = inv
        aff[dst + 1:dst + 2, :] = bet_ref[...] - mean * inv

    @pl.when(pid < B)
    def _():
        b = pid

        @pl.when(b == 0)
        def _():
            st[...] = jnp.zeros_like(st)
            x_copy(0, 0).start()

        for i in range(NCH):
            g = b * NCH + i
            x_copy(g, i % 2).wait()
            if i < NCH - 1:
                x_copy(g + 1, (i + 1) % 2).start()
            else:
                @pl.when(b < B - 1)
                def _():
                    x_copy(g + 1, 0).start()

            xc = iobuf[i % 2]
            hm = jnp.dot(xc.astype(jnp.bfloat16), mb_ref[...],
                         preferred_element_type=jnp.float32)
            st[0:1, :] += jnp.sum(hm, axis=0, keepdims=True)
            st[1:2, :] += jnp.sum(hm * hm, axis=0, keepdims=True)
            hsc[g] = hm.astype(jnp.bfloat16)

    @pl.when(jnp.logical_and(pid >= B, pid < 2 * B))
    def _():
        b = pid - B

        @pl.when(b == 0)
        def _():
            finalize(0, 1, 0)

        sc = aff[0:1, :]
        sh = aff[1:2, :]

        def zchunk(i):
            hb = hsc[b * NCH + i]
            zs = []
            for v in range(V):
                hv = hb[:, v * C:(v + 1) * C].astype(jnp.float32)
                gv = jnp.maximum(hv * sc + sh, 0.0)
                zs.append(jnp.dot(gv.astype(jnp.bfloat16), wt_ref[...],
                                  preferred_element_type=jnp.float32))
            return jnp.concatenate(zs, axis=1).astype(jnp.bfloat16)

        def h2chunk(j):
            rows = atw_ref[j * _CH:(j + 1) * _CH, :]
            sj = (j % 4) * _CH
            h2 = jnp.dot(rows[:, 4:36], zsc[sj:sj + _CH, :],
                         preferred_element_type=jnp.float32)
            if j >= 1:
                sp = ((j - 1) % 4) * _CH
                h2 = h2 + jnp.dot(rows[:, 0:4], zsc[sp + _CH - 4:sp + _CH, :],
                                  preferred_element_type=jnp.float32)
            if j + 1 < NCH:
                sn = ((j + 1) % 4) * _CH
                h2 = h2 + jnp.dot(rows[:, 36:40], zsc[sn:sn + 4, :],
                                  preferred_element_type=jnp.float32)
            st[2:3, :] += jnp.sum(h2, axis=0, keepdims=True)
            st[3:4, :] += jnp.sum(h2 * h2, axis=0, keepdims=True)
            hsc[b * NCH + j] = h2.astype(jnp.bfloat16)

        for i in range(NCH + 1):
            if i < NCH:
                zsc[(i % 4) * _CH:(i % 4) * _CH + _CH, :] = zchunk(i)
            if i >= 1:
                h2chunk(i - 1)

    @pl.when(pid >= 2 * B)
    def _():
        b = pid - 2 * B

        @pl.when(b == 0)
        def _():
            finalize(2, 3, 2)

        sc = aff[2:3, :]
        sh = aff[3:4, :]
        for i in range(NCH):
            g = b * NCH + i
            slot = i % 2
            if i >= 2:
                o_copy(g - 2, slot).wait()
            else:
                @pl.when(b > 0)
                def _():
                    o_copy(g - 2, slot).wait()

            hb = hsc[g]
            os = []
            for v in range(V):
                hv = hb[:, v * C:(v + 1) * C].astype(jnp.float32)
                os.append(jnp.maximum(hv * sc + sh, 0.0))
            iobuf[slot] = jnp.concatenate(os, axis=1)
            o_copy(g, slot).start()

        @pl.when(b == B - 1)
        def _():
            gl = B * NCH - 1
            o_copy(gl - 1, (gl - 1) % 2).wait()
            o_copy(gl, gl % 2).wait()


def kernel(x, adj, edge_importance, W1, b1, W2, b2, W3, b3, Wt, bt, gamma, beta):
    B, T, V, C = x.shape
    f32 = jnp.float32
    VC = V * C
    NCH = T // _CH
    xf = x.reshape(B * NCH, _CH, VC)

    A = adj * edge_importance
    Mb = (jnp.kron(A[0].T, W1) + jnp.kron(A[1].T, W2)
          + jnp.kron(A[2].T, W3)).astype(jnp.bfloat16)
    r = jnp.arange(T)
    w0 = (r // _CH) * _CH - 4
    s = w0[:, None] + jnp.arange(40)[None, :]
    Atw = ((jnp.abs(r[:, None] - s) <= 4) & (s >= 0) & (s < T)
           ).astype(jnp.bfloat16)

    small = lambda shp: pl.BlockSpec(shp, lambda i: (0,) * len(shp))
    p = pl.pallas_call(
        functools.partial(_fused, B=B, T=T, V=V, C=C),
        grid=(3 * B,),
        in_specs=[
            pl.BlockSpec(memory_space=pl.ANY),
            small((VC, VC)), small((C, C)), small((T, 40)),
            small((1, C)), small((1, C)),
        ],
        out_specs=pl.BlockSpec(memory_space=pl.ANY),
        out_shape=jax.ShapeDtypeStruct((B * NCH, _CH, VC), f32),
        scratch_shapes=[
            pltpu.VMEM((B * NCH, _CH, VC), jnp.bfloat16),
            pltpu.VMEM((4 * _CH, VC), jnp.bfloat16),
            pltpu.VMEM((2, _CH, VC), f32),
            pltpu.VMEM((8, VC), f32),
            pltpu.VMEM((4, C), f32),
            pltpu.SemaphoreType.DMA((2,)),
            pltpu.SemaphoreType.DMA((2,)),
        ],
        compiler_params=pltpu.CompilerParams(dimension_semantics=("arbitrary",)),
    )
    out = p(xf, Mb, Wt.astype(jnp.bfloat16), Atw,
            gamma.reshape(1, C), beta.reshape(1, C))
    return out.reshape(B, T, V, C)


# speedup vs baseline: 1.1698x; 1.1698x over previous
import functools

import jax
import jax.numpy as jnp
from jax.experimental import pallas as pl
from jax.experimental.pallas import tpu as pltpu

_EPS = 1e-5
_CH = 32


def _fused(x_hbm, mb_ref, wt_ref, atw_ref, gam_ref, bet_ref,
           out_hbm, hsc, zsc, iobuf, st, aff, insem, outsem,
           *, B, T, V, C):
    pid = pl.program_id(0)
    VC = V * C
    NCH = T // _CH
    n = float(B * T * V)

    def x_copy(g, slot):
        return pltpu.make_async_copy(x_hbm.at[g], iobuf.at[slot], insem.at[slot])

    def o_copy(g, slot):
        return pltpu.make_async_copy(iobuf.at[slot], out_hbm.at[g], outsem.at[slot])

    def fold(row):
        acc = row[:, 0:C]
        for v in range(1, V):
            acc = acc + row[:, v * C:(v + 1) * C]
        return acc

    def finalize(srow, qrow, dst):
        mean = fold(st[srow:srow + 1, :]) / n
        var = fold(st[qrow:qrow + 1, :]) / n - mean * mean
        inv = gam_ref[...] * jax.lax.rsqrt(var + _EPS)
        aff[dst:dst + 1, :] = inv
        aff[dst + 1:dst + 2, :] = bet_ref[...] - mean * inv

    @pl.when(pid < B)
    def _():
        b = pid

        @pl.when(b == 0)
        def _():
            st[...] = jnp.zeros_like(st)
            x_copy(0, 0).start()
            x_copy(1, 1).start()

        for j in range(NCH // 2):
            g0 = b * NCH + 2 * j
            for k in range(2):
                x_copy(g0 + k, k).wait()
                zsc[k * _CH:(k + 1) * _CH, :] = iobuf[k].astype(jnp.bfloat16)
                if 2 * j + 2 + k < NCH:
                    x_copy(g0 + 2 + k, k).start()
                else:
                    @pl.when(b < B - 1)
                    def _():
                        x_copy(g0 + 2 + k, k).start()

            hm = jnp.dot(zsc[0:2 * _CH, :], mb_ref[...],
                         preferred_element_type=jnp.float32)
            st[0:1, :] += jnp.sum(hm, axis=0, keepdims=True)
            st[1:2, :] += jnp.sum(hm * hm, axis=0, keepdims=True)
            hsc[2 * (b * NCH // 2 + j)] = hm[0:_CH, :].astype(jnp.bfloat16)
            hsc[2 * (b * NCH // 2 + j) + 1] = hm[_CH:2 * _CH, :].astype(jnp.bfloat16)

    @pl.when(jnp.logical_and(pid >= B, pid < 2 * B))
    def _():
        b = pid - B

        @pl.when(b == 0)
        def _():
            finalize(0, 1, 0)

        sc = aff[0:1, :]
        sh = aff[1:2, :]

        def zchunk(i):
            hb = hsc[b * NCH + i]
            zs = []
            for v in range(V):
                hv = hb[:, v * C:(v + 1) * C].astype(jnp.float32)
                gv = jnp.maximum(hv * sc + sh, 0.0)
                zs.append(jnp.dot(gv.astype(jnp.bfloat16), wt_ref[...],
                                  preferred_element_type=jnp.float32))
            return jnp.concatenate(zs, axis=1).astype(jnp.bfloat16)

        def h2chunk(j):
            rows = atw_ref[j * _CH:(j + 1) * _CH, :]
            sj = (j % 4) * _CH
            h2 = jnp.dot(rows[:, 4:36], zsc[sj:sj + _CH, :],
                         preferred_element_type=jnp.float32)
            if j >= 1:
                sp = ((j - 1) % 4) * _CH
                h2 = h2 + jnp.dot(rows[:, 0:4], zsc[sp + _CH - 4:sp + _CH, :],
                                  preferred_element_type=jnp.float32)
            if j + 1 < NCH:
                sn = ((j + 1) % 4) * _CH
                h2 = h2 + jnp.dot(rows[:, 36:40], zsc[sn:sn + 4, :],
                                  preferred_element_type=jnp.float32)
            st[2:3, :] += jnp.sum(h2, axis=0, keepdims=True)
            st[3:4, :] += jnp.sum(h2 * h2, axis=0, keepdims=True)
            hsc[b * NCH + j] = h2.astype(jnp.bfloat16)

        for i in range(NCH + 1):
            if i < NCH:
                zsc[(i % 4) * _CH:(i % 4) * _CH + _CH, :] = zchunk(i)
            if i >= 1:
                h2chunk(i - 1)

    @pl.when(pid >= 2 * B)
    def _():
        b = pid - 2 * B

        @pl.when(b == 0)
        def _():
            finalize(2, 3, 2)

        sc = aff[2:3, :]
        sh = aff[3:4, :]
        for i in range(NCH):
            g = b * NCH + i
            slot = i % 2
            if i >= 2:
                o_copy(g - 2, slot).wait()
            else:
                @pl.when(b > 0)
                def _():
                    o_copy(g - 2, slot).wait()

            hb = hsc[g]
            os = []
            for v in range(V):
                hv = hb[:, v * C:(v + 1) * C].astype(jnp.float32)
                os.append(jnp.maximum(hv * sc + sh, 0.0))
            iobuf[slot] = jnp.concatenate(os, axis=1)
            o_copy(g, slot).start()

        @pl.when(b == B - 1)
        def _():
            gl = B * NCH - 1
            o_copy(gl - 1, (gl - 1) % 2).wait()
            o_copy(gl, gl % 2).wait()


def kernel(x, adj, edge_importance, W1, b1, W2, b2, W3, b3, Wt, bt, gamma, beta):
    B, T, V, C = x.shape
    f32 = jnp.float32
    VC = V * C
    NCH = T // _CH
    xf = x.reshape(B * NCH, _CH, VC)

    A = adj * edge_importance
    Mb = (jnp.kron(A[0].T, W1) + jnp.kron(A[1].T, W2)
          + jnp.kron(A[2].T, W3)).astype(jnp.bfloat16)
    r = jnp.arange(T)
    w0 = (r // _CH) * _CH - 4
    s = w0[:, None] + jnp.arange(40)[None, :]
    Atw = ((jnp.abs(r[:, None] - s) <= 4) & (s >= 0) & (s < T)
           ).astype(jnp.bfloat16)

    small = lambda shp: pl.BlockSpec(shp, lambda i: (0,) * len(shp))
    p = pl.pallas_call(
        functools.partial(_fused, B=B, T=T, V=V, C=C),
        grid=(3 * B,),
        in_specs=[
            pl.BlockSpec(memory_space=pl.ANY),
            small((VC, VC)), small((C, C)), small((T, 40)),
            small((1, C)), small((1, C)),
        ],
        out_specs=pl.BlockSpec(memory_space=pl.ANY),
        out_shape=jax.ShapeDtypeStruct((B * NCH, _CH, VC), f32),
        scratch_shapes=[
            pltpu.VMEM((B * NCH, _CH, VC), jnp.bfloat16),
            pltpu.VMEM((4 * _CH, VC), jnp.bfloat16),
            pltpu.VMEM((2, _CH, VC), f32),
            pltpu.VMEM((8, VC), f32),
            pltpu.VMEM((4, C), f32),
            pltpu.SemaphoreType.DMA((2,)),
            pltpu.SemaphoreType.DMA((2,)),
        ],
        compiler_params=pltpu.CompilerParams(dimension_semantics=("arbitrary",)),
    )
    out = p(xf, Mb, Wt.astype(jnp.bfloat16), Atw,
            gamma.reshape(1, C), beta.reshape(1, C))
    return out.reshape(B, T, V, C)


# speedup vs baseline: 3.0840x; 2.6363x over previous
import functools

import jax
import jax.numpy as jnp
from jax.experimental import pallas as pl
from jax.experimental.pallas import tpu as pltpu

_EPS = 1e-5


def _p1_kernel(x_ref, mb_ref, h_ref, st_ref, *, T, V, C):
    xb = x_ref[0]
    hm = jnp.dot(xb.astype(jnp.bfloat16), mb_ref[...],
                 preferred_element_type=jnp.float32)
    h_ref[0] = hm.astype(jnp.bfloat16)
    s = jnp.sum(hm, axis=0, keepdims=True)
    q = jnp.sum(hm * hm, axis=0, keepdims=True)
    part = jnp.concatenate(
        [s, q, jnp.zeros((6, V * C), jnp.float32)], axis=0)

    @pl.when(pl.program_id(0) == 0)
    def _():
        st_ref[...] = jnp.zeros_like(st_ref)

    st_ref[...] += part


def _p2_kernel(h_ref, aff_ref, wt_ref, at_ref, h2_ref, st_ref, *, T, V, C):
    hb = h_ref[0]
    g = jnp.maximum(hb.astype(jnp.float32) * aff_ref[0:1, :]
                    + aff_ref[1:2, :], 0.0)
    zs = []
    for v in range(V):
        zs.append(jnp.dot(g[:, v * C:(v + 1) * C].astype(jnp.bfloat16),
                          wt_ref[...],
                          preferred_element_type=jnp.float32))
    z = jnp.concatenate(zs, axis=1).astype(jnp.bfloat16)
    h2 = jnp.dot(at_ref[...], z,
                 preferred_element_type=jnp.float32)
    h2_ref[0] = h2.astype(jnp.bfloat16)
    s = jnp.sum(h2, axis=0, keepdims=True)
    q = jnp.sum(h2 * h2, axis=0, keepdims=True)
    part = jnp.concatenate(
        [s, q, jnp.zeros((6, V * C), jnp.float32)], axis=0)

    @pl.when(pl.program_id(0) == 0)
    def _():
        st_ref[...] = jnp.zeros_like(st_ref)

    st_ref[...] += part


def _p3_kernel(h2_ref, aff_ref, out_ref, *, T, V, C):
    hb = h2_ref[0]
    out_ref[0] = jnp.maximum(hb.astype(jnp.float32) * aff_ref[0:1, :]
                             + aff_ref[1:2, :], 0.0)


def _bn_affine(st, n, gamma, beta, V, C):
    s = st[0].reshape(V, C).sum(axis=0)
    q = st[1].reshape(V, C).sum(axis=0)
    mean = s / n
    var = q / n - mean * mean
    inv = gamma * jax.lax.rsqrt(var + _EPS)
    aff = jnp.stack([inv, beta - mean * inv])
    return jnp.tile(aff, (1, V))


def kernel(x, adj, edge_importance, W1, b1, W2, b2, W3, b3, Wt, bt, gamma, beta):
    B, T, V, C = x.shape
    f32 = jnp.float32
    bf16 = jnp.bfloat16
    VC = V * C
    n = float(B * T * V)
    xf = x.reshape(B, T, VC)

    A = adj * edge_importance
    Mb = (jnp.kron(A[0].T, W1) + jnp.kron(A[1].T, W2)
          + jnp.kron(A[2].T, W3)).astype(bf16)
    r = jnp.arange(T)
    At = (jnp.abs(r[:, None] - r[None, :]) <= 4).astype(bf16)

    params = pltpu.CompilerParams(dimension_semantics=("arbitrary",))
    small = lambda shp: pl.BlockSpec(shp, lambda b: (0,) * len(shp))
    blk = pl.BlockSpec((1, T, VC), lambda b: (b, 0, 0))

    p1 = pl.pallas_call(
        functools.partial(_p1_kernel, T=T, V=V, C=C),
        grid=(B,),
        in_specs=[blk, small((VC, VC))],
        out_specs=[blk, small((8, VC))],
        out_shape=[jax.ShapeDtypeStruct((B, T, VC), bf16),
                   jax.ShapeDtypeStruct((8, VC), f32)],
        compiler_params=params,
    )
    h, st1 = p1(xf, Mb)
    aff1 = _bn_affine(st1, n, gamma, beta, V, C)

    p2 = pl.pallas_call(
        functools.partial(_p2_kernel, T=T, V=V, C=C),
        grid=(B,),
        in_specs=[blk, small((2, VC)), small((C, C)), small((T, T))],
        out_specs=[blk, small((8, VC))],
        out_shape=[jax.ShapeDtypeStruct((B, T, VC), bf16),
                   jax.ShapeDtypeStruct((8, VC), f32)],
        compiler_params=params,
    )
    h2, st2 = p2(h, aff1, Wt.astype(bf16), At)
    aff2 = _bn_affine(st2, n, gamma, beta, V, C)

    p3 = pl.pallas_call(
        functools.partial(_p3_kernel, T=T, V=V, C=C),
        grid=(B,),
        in_specs=[blk, small((2, VC))],
        out_specs=blk,
        out_shape=jax.ShapeDtypeStruct((B, T, VC), f32),
        compiler_params=params,
    )
    out = p3(h2, aff2)
    return out.reshape(B, T, V, C)
